# baseline (speedup 1.0000x reference)
import functools

import jax
import jax.numpy as jnp
from jax.experimental import pallas as pl
from jax.experimental.pallas import tpu as pltpu


def _plane_accum_body(inv_hw, num_k, x_ref, o_ref):
    s = jnp.sum(x_ref[...], axis=0, dtype=jnp.float32)
    if num_k == 1:
        o_ref[...] = (s * inv_hw).astype(o_ref.dtype)
        return

    k = pl.program_id(1)

    @pl.when(k == 0)
    def _():
        o_ref[...] = s.astype(o_ref.dtype)

    @pl.when(jnp.logical_and(k > 0, k < num_k - 1))
    def _():
        o_ref[...] += s.astype(o_ref.dtype)

    @pl.when(k == num_k - 1)
    def _():
        o_ref[...] = ((o_ref[...] + s) * inv_hw).astype(o_ref.dtype)


def _rowsum_body(inv_hw, x_ref, o_ref):
    s = jnp.sum(x_ref[...], axis=-1, keepdims=True, dtype=jnp.float32)
    o_ref[...] = (s * inv_hw).astype(o_ref.dtype)


def kernel(x):
    N, C, H, W = x.shape
    HW = H * W

    if HW == 0:
        return jnp.full((N, C), jnp.nan, dtype=x.dtype)
    if N * C == 0:
        return jnp.zeros((N, C), dtype=x.dtype)

    inv_hw = 1.0 / float(HW)
    itemsize = jnp.dtype(x.dtype).itemsize

    if N % 8 == 0 and C % 128 == 0:
        xt = x.transpose(2, 3, 0, 1).reshape(HW, N, C)
        c_blk = C
        num_c = 1
        for cand_blk in (512, 256, 128):
            if C % cand_blk == 0 and C // cand_blk >= 2:
                c_blk = cand_blk
                num_c = C // cand_blk
                break

        cap_kb = max(1, (8 * 1024 * 1024) // (N * c_blk * itemsize))
        kb = 1
        for d in range(min(HW, cap_kb), 0, -1):
            if HW % d == 0:
                kb = d
                break
        num_k = HW // kb

        out = pl.pallas_call(
            functools.partial(_plane_accum_body, inv_hw, num_k),
            out_shape=jax.ShapeDtypeStruct((N, C), x.dtype),
            grid=(num_c, num_k),
            in_specs=[pl.BlockSpec((kb, N, c_blk), lambda j, k: (k, 0, j))],
            out_specs=pl.BlockSpec((N, c_blk), lambda j, k: (0, j)),
            compiler_params=pltpu.CompilerParams(
                dimension_semantics=("parallel", "arbitrary"),
                vmem_limit_bytes=48 * 1024 * 1024,
            ),
            cost_estimate=pl.CostEstimate(
                flops=N * C * HW,
                transcendentals=0,
                bytes_accessed=(N * C * HW + N * C) * itemsize,
            ),
        )(xt)
        return out

    R = N * C
    x2 = x.reshape(R, HW)
    hw128 = ((HW + 127) // 128) * 128
    row_tile = max(8, min(R, (6 * 1024 * 1024) // (hw128 * itemsize) // 8 * 8))
    num_row = pl.cdiv(R, row_tile)
    out = pl.pallas_call(
        functools.partial(_rowsum_body, inv_hw),
        out_shape=jax.ShapeDtypeStruct((R, 1), x.dtype),
        grid=(num_row,),
        in_specs=[pl.BlockSpec((row_tile, HW), lambda r: (r, 0))],
        out_specs=pl.BlockSpec((row_tile, 1), lambda r: (r, 0)),
        compiler_params=pltpu.CompilerParams(
            dimension_semantics=("parallel",),
            vmem_limit_bytes=48 * 1024 * 1024,
        ),
    )(x2)
    return out[:, 0].reshape(N, C)


# speedup vs baseline: 1.1256x; 1.1256x over previous
import functools

import jax
import jax.numpy as jnp
from jax.experimental import pallas as pl
from jax.experimental.pallas import tpu as pltpu


def _plane_accum_body(inv_hw, num_k, x_ref, o_ref):
    s = jnp.sum(x_ref[...], axis=0, dtype=jnp.float32)
    if num_k == 1:
        o_ref[...] = (s * inv_hw).astype(o_ref.dtype)
        return

    k = pl.program_id(1)

    @pl.when(k == 0)
    def _():
        o_ref[...] = s.astype(o_ref.dtype)

    @pl.when(jnp.logical_and(k > 0, k < num_k - 1))
    def _():
        o_ref[...] += s.astype(o_ref.dtype)

    @pl.when(k == num_k - 1)
    def _():
        o_ref[...] = ((o_ref[...] + s) * inv_hw).astype(o_ref.dtype)


def _rowsum_body(inv_hw, x_ref, o_ref):
    s = jnp.sum(x_ref[...], axis=-1, keepdims=True, dtype=jnp.float32)
    o_ref[...] = (s * inv_hw).astype(o_ref.dtype)


def kernel(x):
    N, C, H, W = x.shape
    HW = H * W

    if HW == 0:
        return jnp.full((N, C), jnp.nan, dtype=x.dtype)
    if N * C == 0:
        return jnp.zeros((N, C), dtype=x.dtype)

    inv_hw = 1.0 / float(HW)
    itemsize = jnp.dtype(x.dtype).itemsize

    if N % 8 == 0 and C % 128 == 0:
        xt = x.transpose(2, 3, 0, 1).reshape(HW, N, C)
        c_blk = C
        num_c = 1
        for cand_blk in (1024, 512, 256, 128):
            if C % cand_blk == 0 and C // cand_blk >= 2:
                c_blk = cand_blk
                num_c = C // cand_blk
                break

        cap_kb = max(1, (8 * 1024 * 1024) // (N * c_blk * itemsize))
        kb = 1
        for d in range(min(HW, cap_kb), 0, -1):
            if HW % d == 0:
                kb = d
                break
        num_k = HW // kb

        out = pl.pallas_call(
            functools.partial(_plane_accum_body, inv_hw, num_k),
            out_shape=jax.ShapeDtypeStruct((N, C), x.dtype),
            grid=(num_c, num_k),
            in_specs=[pl.BlockSpec((kb, N, c_blk), lambda j, k: (k, 0, j))],
            out_specs=pl.BlockSpec((N, c_blk), lambda j, k: (0, j)),
            compiler_params=pltpu.CompilerParams(
                dimension_semantics=("parallel", "arbitrary"),
                vmem_limit_bytes=48 * 1024 * 1024,
            ),
            cost_estimate=pl.CostEstimate(
                flops=N * C * HW,
                transcendentals=0,
                bytes_accessed=(N * C * HW + N * C) * itemsize,
            ),
        )(xt)
        return out

    R = N * C
    x2 = x.reshape(R, HW)
    hw128 = ((HW + 127) // 128) * 128
    row_tile = max(8, min(R, (6 * 1024 * 1024) // (hw128 * itemsize) // 8 * 8))
    num_row = pl.cdiv(R, row_tile)
    out = pl.pallas_call(
        functools.partial(_rowsum_body, inv_hw),
        out_shape=jax.ShapeDtypeStruct((R, 1), x.dtype),
        grid=(num_row,),
        in_specs=[pl.BlockSpec((row_tile, HW), lambda r: (r, 0))],
        out_specs=pl.BlockSpec((row_tile, 1), lambda r: (r, 0)),
        compiler_params=pltpu.CompilerParams(
            dimension_semantics=("parallel",),
            vmem_limit_bytes=48 * 1024 * 1024,
        ),
    )(x2)
    return out[:, 0].reshape(N, C)
